# baseline (speedup 1.0000x reference)
import functools

import jax
import jax.numpy as jnp
from jax import lax
from jax.experimental import pallas as pl
from jax.experimental.pallas import tpu as pltpu
from jax.experimental.pallas import tpu_sc as plsc

NF = 8388608
N_CELLS = 4096
N_REGIONS = 1024
NB = N_CELLS * N_REGIONS

NC = 2
NS = 16

NPASS = 2
Q = NB // (NC * NPASS)
DUMMY = 65536
FPT = NF // NS
C = 4096
K = FPT // C
ROWS = C // 128
QP = Q // NS


def _sc_body(frag, regions2, cells2, baseline, lib,
             count_out, rb_out, cl_out,
             hist, idxbuf, cbuf, zbuf, ones,
             sem_in, sem_s0, sem_s1, sem_g):
    c = lax.axis_index("c")
    s = lax.axis_index("s")

    zv = jnp.zeros((16,), jnp.float32)
    ov = jnp.ones((16,), jnp.float32)
    iv = lax.iota(jnp.int32, 16)

    def _z16(i, _):
        zbuf[pl.ds(i * 16, 16)] = zv
        return 0
    lax.fori_loop(0, C // 16, _z16, 0)
    for jj in range(8):
        ones[pl.ds(jj * 16, 16)] = ov

    dvec0 = (iv * 4096 + s * 256) & (DUMMY - 1)

    for p in range(NPASS):
        r = NPASS * p + c
        base = r * Q

        hoff = s * QP

        def _zh(i, _):
            pltpu.sync_copy(zbuf, hist.at[pl.ds(hoff + i * C, C)])
            return 0
        lax.fori_loop(0, QP // C, _zh, 0)
        plsc.subcore_barrier()

        pltpu.async_copy(frag.at[pl.ds(s * FPT, C)], idxbuf.at[0], sem_in)

        def _drain0(i, _):
            pltpu.make_async_copy(frag.at[pl.ds(0, 128)], idxbuf.at[0, pl.ds(0, 128)], sem_s0).wait()
            return 0

        def _drain1(i, _):
            pltpu.make_async_copy(frag.at[pl.ds(0, 128)], idxbuf.at[0, pl.ds(0, 128)], sem_s1).wait()
            return 0

        def _chunk(k, carry):
            dvec, n1, n2 = carry
            kb = lax.rem(k, 2)
            pltpu.make_async_copy(frag.at[pl.ds(0, C)], idxbuf.at[0],
                                  sem_in).wait()

            @pl.when(k + 1 < K)
            def _():
                pltpu.async_copy(frag.at[pl.ds(s * FPT + (k + 1) * C, C)],
                                 idxbuf.at[1 - kb], sem_in)

            @pl.when(kb == 0)
            def _():
                lax.fori_loop(0, n2, _drain0, 0)

            @pl.when(kb == 1)
            def _():
                lax.fori_loop(0, n2, _drain1, 0)

            def _row(j, off):
                for jj in range(8):
                    v = idxbuf[kb, pl.ds(j * 128 + jj * 16, 16)]
                    loc = v - base
                    m = jnp.logical_and(loc >= 0, loc < Q)
                    pos = off + plsc.cumsum(m.astype(jnp.int32)) - 1
                    plsc.store_scatter(cbuf.at[kb], [pos >> 7, pos & 127],
                                       loc, mask=m)
                    off = off + plsc.all_reduce_population_count(m)
                return off
            off = lax.fori_loop(0, ROWS, _row, jnp.zeros((16,), jnp.int32))
            offs = jnp.max(off)

            dvec = (dvec + 131) & (DUMMY - 1)
            mall = iv >= 0
            for jj in range(8):
                p = offs + jj * 16 + iv
                dval = Q + ((dvec + jj * 272) & (DUMMY - 1))
                plsc.store_scatter(cbuf.at[kb], [p >> 7, p & 127], dval,
                                   mask=mall)
            nrows = (offs + 127) >> 7

            def _fire0(j, _):
                pltpu.async_copy(ones, hist.at[cbuf.at[kb, j]], sem_s0,
                                 add=True)
                return 0

            def _fire1(j, _):
                pltpu.async_copy(ones, hist.at[cbuf.at[kb, j]], sem_s1,
                                 add=True)
                return 0

            @pl.when(kb == 0)
            def _():
                lax.fori_loop(0, nrows, _fire0, 0)

            @pl.when(kb == 1)
            def _():
                lax.fori_loop(0, nrows, _fire1, 0)
            return (dvec, nrows, n1)

        dvec0, nlast, nprev = lax.fori_loop(
            0, K, _chunk, (dvec0, jnp.int32(0), jnp.int32(0)))

        lax.fori_loop(0, nprev, _drain0, 0)
        lax.fori_loop(0, nlast, _drain1, 0)
        plsc.subcore_barrier()
        pltpu.sync_copy(hist.at[pl.ds(s * QP, QP)],
                        count_out.at[pl.ds(base + s * QP, QP)])

    @pl.when(jnp.logical_and(c == 0, s == 1))
    def _():
        pltpu.sync_copy(regions2, cbuf.at[0, pl.ds(0, 8)])
        for j in range(8):
            pltpu.async_copy(baseline.at[cbuf.at[0, j]],
                             zbuf.at[pl.ds(j * 128, 128)], sem_g).wait()
        pltpu.sync_copy(zbuf.at[pl.ds(0, 1024)], rb_out)

    @pl.when(jnp.logical_and(c == 1, s == 1))
    def _():
        pltpu.sync_copy(cells2, cbuf.at[0, pl.ds(0, 32)])
        for j in range(32):
            pltpu.async_copy(lib.at[cbuf.at[0, j]],
                             zbuf.at[pl.ds(j * 128, 128)], sem_g).wait()
        pltpu.sync_copy(zbuf, cl_out)


@functools.cache
def _sc_histogram():
    return functools.partial(
        pl.kernel,
        out_type=(
            jax.ShapeDtypeStruct((NB,), jnp.float32),
            jax.ShapeDtypeStruct((N_REGIONS,), jnp.float32),
            jax.ShapeDtypeStruct((N_CELLS,), jnp.float32),
        ),
        mesh=plsc.VectorSubcoreMesh(core_axis_name="c", subcore_axis_name="s",
                                    num_cores=NC, num_subcores=NS),
        compiler_params=pltpu.CompilerParams(needs_layout_passes=False),
        scratch_types=[
            pltpu.VMEM_SHARED((Q + DUMMY,), jnp.float32),
            pltpu.VMEM((2, C), jnp.int32),
            pltpu.VMEM((2, ROWS + 1, 128), jnp.int32),
            pltpu.VMEM((C,), jnp.float32),
            pltpu.VMEM((128,), jnp.float32),
            pltpu.SemaphoreType.DMA,
            pltpu.SemaphoreType.DMA,
            pltpu.SemaphoreType.DMA,
            pltpu.SemaphoreType.DMA,
        ],
    )(_sc_body)


BLK = 512


def _gammaln1p(c):
    x = c + 1.0
    p = (x * (x + 1.0) * (x + 2.0) * (x + 3.0)
         * (x + 4.0) * (x + 5.0) * (x + 6.0) * (x + 7.0))
    z = x + 8.0
    zi = 1.0 / z
    zi2 = zi * zi
    series = zi * (1.0 / 12.0 - zi2 * (1.0 / 360.0 - zi2 * (1.0 / 1260.0)))
    lg = (z - 0.5) * jnp.log(z) - z + 0.9189385332046727 + series
    return lg - jnp.log(p)


def _tc_body(count_ref, cl_ref, rb_ref, out_ref):
    cnt = count_ref[...]
    logits = cl_ref[...] + rb_ref[...]
    rate = jnp.exp(logits)
    out_ref[...] = cnt * logits - rate - _gammaln1p(cnt)


_tc_likelihood = pl.pallas_call(
    _tc_body,
    grid=(N_CELLS // BLK,),
    in_specs=[
        pl.BlockSpec((BLK, N_REGIONS), lambda i: (i, 0)),
        pl.BlockSpec((BLK, 1), lambda i: (i, 0)),
        pl.BlockSpec((1, N_REGIONS), lambda i: (0, 0)),
    ],
    out_specs=pl.BlockSpec((BLK, N_REGIONS), lambda i: (i, 0)),
    out_shape=jax.ShapeDtypeStruct((N_CELLS, N_REGIONS), jnp.float32),
)


def kernel(local_cellxregion_ix, regions_oi, cells_oi, baseline_weight, lib):
    regions2 = regions_oi.reshape(8, 128)
    cells2 = cells_oi.reshape(32, 128)
    baseline_flat = baseline_weight.reshape(-1)
    count, rb, cl = _sc_histogram()(local_cellxregion_ix, regions2, cells2,
                                    baseline_flat, lib)
    count2 = count.reshape(N_CELLS, N_REGIONS)
    cl2 = cl.reshape(N_CELLS, 1)
    rb2 = rb.reshape(1, N_REGIONS)
    return _tc_likelihood(count2, cl2, rb2)


# speedup vs baseline: 2.2547x; 2.2547x over previous
import functools

import jax
import jax.numpy as jnp
from jax import lax
from jax.experimental import pallas as pl
from jax.experimental.pallas import tpu as pltpu
from jax.experimental.pallas import tpu_sc as plsc

NF = 8388608
N_CELLS = 4096
N_REGIONS = 1024
NB = N_CELLS * N_REGIONS

NC = 2
NS = 16

H = NB // NC
HW = H // 2
DUMMY = 65536
FPT = NF // NS
C = 4096
K = FPT // C
ROWS = C // 128
HPW = HW // NS


def _sc_body(frag, regions2, cells2, baseline, lib,
             count_out, rb_out, cl_out,
             hist, idxbuf, cbuf, vbuf, zbuf, pbuf,
             sem_in, sem_s0, sem_s1, sem_g):
    c = lax.axis_index("c")
    s = lax.axis_index("s")
    base = c * H

    zvi = jnp.zeros((16,), jnp.int32)
    iv = lax.iota(jnp.int32, 16)

    def _z16(i, _):
        idxbuf[0, pl.ds(i * 16, 16)] = zvi
        return 0
    lax.fori_loop(0, C // 16, _z16, 0)

    hoff = s * HPW

    def _zh(i, _):
        pltpu.sync_copy(idxbuf.at[0], hist.at[pl.ds(hoff + i * C, C)])
        return 0
    lax.fori_loop(0, HPW // C, _zh, 0)
    plsc.subcore_barrier()

    dvec0 = (iv * 4096 + s * 256) & (DUMMY - 1)
    pltpu.async_copy(frag.at[pl.ds(s * FPT, C)], idxbuf.at[0], sem_in)

    def _chunk(k, dvec):
        kb = lax.rem(k, 2)
        pltpu.make_async_copy(frag.at[pl.ds(0, C)], idxbuf.at[0],
                              sem_in).wait()

        @pl.when(k + 1 < K)
        def _():
            pltpu.async_copy(frag.at[pl.ds(s * FPT + (k + 1) * C, C)],
                             idxbuf.at[1 - kb], sem_in)

        @pl.when(jnp.logical_and(k >= 2, kb == 0))
        def _():
            pltpu.make_async_copy(frag.at[pl.ds(0, C)], idxbuf.at[0],
                                  sem_s0).wait()

        @pl.when(jnp.logical_and(k >= 2, kb == 1))
        def _():
            pltpu.make_async_copy(frag.at[pl.ds(0, C)], idxbuf.at[0],
                                  sem_s1).wait()

        def _row(j, dv):
            for jj in range(8):
                v = idxbuf[kb, pl.ds(j * 128 + jj * 16, 16)]
                loc = v - base
                m = jnp.logical_and(loc >= 0, loc < H)
                dv = (dv + 61) & (DUMMY - 1)
                word = jnp.where(m, loc >> 1, HW + dv)
                val = jnp.where((v & 1) == 1, 1 << 16, 1)
                cbuf[kb, j, pl.ds(jj * 16, 16)] = word
                vbuf[kb, j, pl.ds(jj * 16, 16)] = val
            return dv
        dvec = lax.fori_loop(0, ROWS, _row, dvec)

        def _fire0(j, _):
            pltpu.async_copy(vbuf.at[kb, j], hist.at[cbuf.at[kb, j]],
                             sem_s0, add=True)
            return 0

        def _fire1(j, _):
            pltpu.async_copy(vbuf.at[kb, j], hist.at[cbuf.at[kb, j]],
                             sem_s1, add=True)
            return 0

        @pl.when(kb == 0)
        def _():
            lax.fori_loop(0, ROWS, _fire0, 0)

        @pl.when(kb == 1)
        def _():
            lax.fori_loop(0, ROWS, _fire1, 0)
        return dvec

    lax.fori_loop(0, K, _chunk, dvec0)

    pltpu.make_async_copy(frag.at[pl.ds(0, C)], idxbuf.at[0], sem_s0).wait()
    pltpu.make_async_copy(frag.at[pl.ds(0, C)], idxbuf.at[0], sem_s1).wait()
    plsc.subcore_barrier()

    pltpu.sync_copy(hist.at[pl.ds(hoff, HPW)],
                    count_out.at[pl.ds(c * HW + hoff, HPW)])

    @pl.when(jnp.logical_and(c == 0, s == 1))
    def _():
        pltpu.sync_copy(regions2, cbuf.at[0, pl.ds(0, 8)])
        for j in range(8):
            pltpu.async_copy(baseline.at[cbuf.at[0, j]],
                             zbuf.at[pl.ds(j * 128, 128)], sem_g).wait()
        pltpu.sync_copy(zbuf.at[pl.ds(0, 1024)], rb_out)

    @pl.when(jnp.logical_and(c == 1, s == 1))
    def _():
        pltpu.sync_copy(cells2, cbuf.at[0, pl.ds(0, 32)])
        for j in range(32):
            pltpu.async_copy(lib.at[cbuf.at[0, j]],
                             zbuf.at[pl.ds(j * 128, 128)], sem_g).wait()
        pltpu.sync_copy(zbuf, cl_out)


@functools.cache
def _sc_histogram():
    return functools.partial(
        pl.kernel,
        out_type=(
            jax.ShapeDtypeStruct((NB // 2,), jnp.int32),
            jax.ShapeDtypeStruct((N_REGIONS,), jnp.float32),
            jax.ShapeDtypeStruct((N_CELLS,), jnp.float32),
        ),
        mesh=plsc.VectorSubcoreMesh(core_axis_name="c", subcore_axis_name="s",
                                    num_cores=NC, num_subcores=NS),
        compiler_params=pltpu.CompilerParams(needs_layout_passes=False),
        scratch_types=[
            pltpu.VMEM_SHARED((HW + DUMMY,), jnp.int32),
            pltpu.VMEM((2, C), jnp.int32),
            pltpu.VMEM((2, ROWS, 128), jnp.int32),
            pltpu.VMEM((2, ROWS, 128), jnp.int32),
            pltpu.VMEM((C,), jnp.float32),
            pltpu.VMEM((2 * C,), jnp.int16),
            pltpu.SemaphoreType.DMA,
            pltpu.SemaphoreType.DMA,
            pltpu.SemaphoreType.DMA,
            pltpu.SemaphoreType.DMA,
        ],
    )(_sc_body)


BLK = 512


def _gammaln1p(c):
    x = c + 1.0
    p = (x * (x + 1.0) * (x + 2.0) * (x + 3.0)
         * (x + 4.0) * (x + 5.0) * (x + 6.0) * (x + 7.0))
    z = x + 8.0
    zi = 1.0 / z
    zi2 = zi * zi
    series = zi * (1.0 / 12.0 - zi2 * (1.0 / 360.0 - zi2 * (1.0 / 1260.0)))
    lg = (z - 0.5) * jnp.log(z) - z + 0.9189385332046727 + series
    return lg - jnp.log(p)


def _tc_body(count_ref, cl_ref, rb_ref, out_ref):
    cnt = count_ref[...].astype(jnp.float32)
    logits = cl_ref[...] + rb_ref[...]
    rate = jnp.exp(logits)
    out_ref[...] = cnt * logits - rate - _gammaln1p(cnt)


_tc_likelihood = pl.pallas_call(
    _tc_body,
    grid=(N_CELLS // BLK,),
    in_specs=[
        pl.BlockSpec((BLK, N_REGIONS), lambda i: (i, 0)),
        pl.BlockSpec((BLK, 1), lambda i: (i, 0)),
        pl.BlockSpec((1, N_REGIONS), lambda i: (0, 0)),
    ],
    out_specs=pl.BlockSpec((BLK, N_REGIONS), lambda i: (i, 0)),
    out_shape=jax.ShapeDtypeStruct((N_CELLS, N_REGIONS), jnp.float32),
)


def kernel(local_cellxregion_ix, regions_oi, cells_oi, baseline_weight, lib):
    regions2 = regions_oi.reshape(8, 128)
    cells2 = cells_oi.reshape(32, 128)
    baseline_flat = baseline_weight.reshape(-1)
    count, rb, cl = _sc_histogram()(local_cellxregion_ix, regions2, cells2,
                                    baseline_flat, lib)
    count16 = jax.lax.bitcast_convert_type(count, jnp.int16).reshape(NB)
    count2 = count16.reshape(N_CELLS, N_REGIONS)
    cl2 = cl.reshape(N_CELLS, 1)
    rb2 = rb.reshape(1, N_REGIONS)
    return _tc_likelihood(count2, cl2, rb2)


# speedup vs baseline: 2.4933x; 1.1058x over previous
import functools

import jax
import jax.numpy as jnp
from jax import lax
from jax.experimental import pallas as pl
from jax.experimental.pallas import tpu as pltpu
from jax.experimental.pallas import tpu_sc as plsc

NF = 8388608
N_CELLS = 4096
N_REGIONS = 1024
NB = N_CELLS * N_REGIONS

NC = 2
NS = 16

H = NB // NC
HW = H // 2
DUMMY = 65536
FPT = NF // NS
C = 4096
K = FPT // C
ROWS = C // 128
HPW = HW // NS


def _sc_body(frag, regions2, cells2, baseline, lib,
             count_out, rb_out, cl_out,
             hist, idxbuf, cbuf, vbuf, zbuf, pbuf,
             sem_in, sem_s0, sem_s1, sem_g):
    c = lax.axis_index("c")
    s = lax.axis_index("s")
    base = c * H

    zvi = jnp.zeros((16,), jnp.int32)
    iv = lax.iota(jnp.int32, 16)

    def _z16(i, _):
        idxbuf[0, pl.ds(i * 16, 16)] = zvi
        return 0
    lax.fori_loop(0, C // 16, _z16, 0)

    hoff = s * HPW

    def _zh(i, _):
        pltpu.sync_copy(idxbuf.at[0], hist.at[pl.ds(hoff + i * C, C)])
        return 0
    lax.fori_loop(0, HPW // C, _zh, 0)
    plsc.subcore_barrier()

    dvec0 = (iv * 4096 + s * 256) & (DUMMY - 1)
    pltpu.async_copy(frag.at[pl.ds(s * FPT, C)], idxbuf.at[0], sem_in)

    def _chunk(k, dvec):
        kb = lax.rem(k, 2)
        pltpu.make_async_copy(frag.at[pl.ds(0, C)], idxbuf.at[0],
                              sem_in).wait()

        @pl.when(k + 1 < K)
        def _():
            pltpu.async_copy(frag.at[pl.ds(s * FPT + (k + 1) * C, C)],
                             idxbuf.at[1 - kb], sem_in)

        @pl.when(jnp.logical_and(k >= 2, kb == 0))
        def _():
            pltpu.make_async_copy(frag.at[pl.ds(0, C)], idxbuf.at[0],
                                  sem_s0).wait()

        @pl.when(jnp.logical_and(k >= 2, kb == 1))
        def _():
            pltpu.make_async_copy(frag.at[pl.ds(0, C)], idxbuf.at[0],
                                  sem_s1).wait()

        def _row(j, dv):
            for jj in range(8):
                v = idxbuf[kb, pl.ds(j * 128 + jj * 16, 16)]
                loc = v - base
                m = jnp.logical_and(loc >= 0, loc < H)
                dv = (dv + 61) & (DUMMY - 1)
                word = jnp.where(m, loc >> 1, HW + dv)
                val = jnp.where((v & 1) == 1, 1 << 16, 1)
                cbuf[kb, j, pl.ds(jj * 16, 16)] = word
                vbuf[kb, j, pl.ds(jj * 16, 16)] = val
            return dv
        dvec = lax.fori_loop(0, ROWS, _row, dvec)

        def _fire0(j, _):
            pltpu.async_copy(vbuf.at[kb, j], hist.at[cbuf.at[kb, j]],
                             sem_s0, add=True)
            return 0

        def _fire1(j, _):
            pltpu.async_copy(vbuf.at[kb, j], hist.at[cbuf.at[kb, j]],
                             sem_s1, add=True)
            return 0

        @pl.when(kb == 0)
        def _():
            lax.fori_loop(0, ROWS, _fire0, 0)

        @pl.when(kb == 1)
        def _():
            lax.fori_loop(0, ROWS, _fire1, 0)
        return dvec

    lax.fori_loop(0, K, _chunk, dvec0)

    pltpu.make_async_copy(frag.at[pl.ds(0, C)], idxbuf.at[0], sem_s0).wait()
    pltpu.make_async_copy(frag.at[pl.ds(0, C)], idxbuf.at[0], sem_s1).wait()
    plsc.subcore_barrier()

    def _dump(i, _):
        pltpu.sync_copy(hist.at[pl.ds(hoff + i * C, C)], idxbuf.at[0])

        def _unpack(t, _):
            w = idxbuf[0, pl.ds(t * 16, 16)]
            pbuf[pl.ds(t * 32, 32)] = plsc.bitcast(w, jnp.int16)
            return 0
        lax.fori_loop(0, C // 16, _unpack, 0)
        pltpu.sync_copy(pbuf,
                        count_out.at[pl.ds(2 * (c * HW + hoff + i * C),
                                           2 * C)])
        return 0
    lax.fori_loop(0, HPW // C, _dump, 0)

    @pl.when(jnp.logical_and(c == 0, s == 1))
    def _():
        pltpu.sync_copy(regions2, cbuf.at[0, pl.ds(0, 8)])
        for j in range(8):
            pltpu.async_copy(baseline.at[cbuf.at[0, j]],
                             zbuf.at[pl.ds(j * 128, 128)], sem_g).wait()
        pltpu.sync_copy(zbuf.at[pl.ds(0, 1024)], rb_out)

    @pl.when(jnp.logical_and(c == 1, s == 1))
    def _():
        pltpu.sync_copy(cells2, cbuf.at[0, pl.ds(0, 32)])
        for j in range(32):
            pltpu.async_copy(lib.at[cbuf.at[0, j]],
                             zbuf.at[pl.ds(j * 128, 128)], sem_g).wait()
        pltpu.sync_copy(zbuf, cl_out)


@functools.cache
def _sc_histogram():
    return functools.partial(
        pl.kernel,
        out_type=(
            jax.ShapeDtypeStruct((NB,), jnp.int16),
            jax.ShapeDtypeStruct((N_REGIONS,), jnp.float32),
            jax.ShapeDtypeStruct((N_CELLS,), jnp.float32),
        ),
        mesh=plsc.VectorSubcoreMesh(core_axis_name="c", subcore_axis_name="s",
                                    num_cores=NC, num_subcores=NS),
        compiler_params=pltpu.CompilerParams(needs_layout_passes=False),
        scratch_types=[
            pltpu.VMEM_SHARED((HW + DUMMY,), jnp.int32),
            pltpu.VMEM((2, C), jnp.int32),
            pltpu.VMEM((2, ROWS, 128), jnp.int32),
            pltpu.VMEM((2, ROWS, 128), jnp.int32),
            pltpu.VMEM((C,), jnp.float32),
            pltpu.VMEM((2 * C,), jnp.int16),
            pltpu.SemaphoreType.DMA,
            pltpu.SemaphoreType.DMA,
            pltpu.SemaphoreType.DMA,
            pltpu.SemaphoreType.DMA,
        ],
    )(_sc_body)


BLK = 1024


def _gammaln1p(c):
    x = c + 1.0
    p = (x * (x + 1.0) * (x + 2.0) * (x + 3.0)
         * (x + 4.0) * (x + 5.0) * (x + 6.0) * (x + 7.0))
    z = x + 8.0
    zi = 1.0 / z
    zi2 = zi * zi
    series = zi * (1.0 / 12.0 - zi2 * (1.0 / 360.0 - zi2 * (1.0 / 1260.0)))
    lg = (z - 0.5) * jnp.log(z) - z + 0.9189385332046727 + series
    return lg - jnp.log(p)


def _tc_body(count_ref, cl_ref, rb_ref, out_ref):
    cnt = count_ref[...].astype(jnp.float32)
    logits = cl_ref[...] + rb_ref[...]
    rate = jnp.exp(logits)
    out_ref[...] = cnt * logits - rate - _gammaln1p(cnt)


_tc_likelihood = pl.pallas_call(
    _tc_body,
    grid=(N_CELLS // BLK,),
    in_specs=[
        pl.BlockSpec((BLK, N_REGIONS), lambda i: (i, 0)),
        pl.BlockSpec((BLK, 1), lambda i: (i, 0)),
        pl.BlockSpec((1, N_REGIONS), lambda i: (0, 0)),
    ],
    out_specs=pl.BlockSpec((BLK, N_REGIONS), lambda i: (i, 0)),
    out_shape=jax.ShapeDtypeStruct((N_CELLS, N_REGIONS), jnp.float32),
)


def kernel(local_cellxregion_ix, regions_oi, cells_oi, baseline_weight, lib):
    regions2 = regions_oi.reshape(8, 128)
    cells2 = cells_oi.reshape(32, 128)
    baseline_flat = baseline_weight.reshape(-1)
    count, rb, cl = _sc_histogram()(local_cellxregion_ix, regions2, cells2,
                                    baseline_flat, lib)
    count2 = count.reshape(N_CELLS, N_REGIONS)
    cl2 = cl.reshape(N_CELLS, 1)
    rb2 = rb.reshape(1, N_REGIONS)
    return _tc_likelihood(count2, cl2, rb2)


# speedup vs baseline: 2.6447x; 1.0607x over previous
import functools

import jax
import jax.numpy as jnp
from jax import lax
from jax.experimental import pallas as pl
from jax.experimental.pallas import tpu as pltpu
from jax.experimental.pallas import tpu_sc as plsc

NF = 8388608
N_CELLS = 4096
N_REGIONS = 1024
NB = N_CELLS * N_REGIONS

NC = 2
NS = 16

H = NB // NC
HW = H // 2
DUMMY = 65536
FPT = NF // NS
C = 4096
K = FPT // C
ROWS = C // 128
HPW = HW // NS


def _sc_body(frag, regions2, cells2, baseline, lib,
             count_out, rb_out, cl_out,
             hist, idxbuf, cbuf, vbuf, zbuf, pbufa, pbufb,
             sem_in, sem_s0, sem_s1, sem_g):
    c = lax.axis_index("c")
    s = lax.axis_index("s")
    base = c * H

    zvi = jnp.zeros((16,), jnp.int32)
    iv = lax.iota(jnp.int32, 16)

    def _z16(i, _):
        idxbuf[0, pl.ds(i * 16, 16)] = zvi
        return 0
    lax.fori_loop(0, C // 16, _z16, 0)

    hoff = s * HPW

    def _zh(i, _):
        pltpu.sync_copy(idxbuf.at[0], hist.at[pl.ds(hoff + i * C, C)])
        return 0
    lax.fori_loop(0, HPW // C, _zh, 0)
    plsc.subcore_barrier()

    dvec0 = (iv * 4096 + s * 256) & (DUMMY - 1)
    pltpu.async_copy(frag.at[pl.ds(s * FPT, C)], idxbuf.at[0], sem_in)

    def _chunk(k, dvec):
        kb = lax.rem(k, 2)
        pltpu.make_async_copy(frag.at[pl.ds(0, C)], idxbuf.at[0],
                              sem_in).wait()

        @pl.when(k + 1 < K)
        def _():
            pltpu.async_copy(frag.at[pl.ds(s * FPT + (k + 1) * C, C)],
                             idxbuf.at[1 - kb], sem_in)

        @pl.when(jnp.logical_and(k >= 2, kb == 0))
        def _():
            pltpu.make_async_copy(frag.at[pl.ds(0, C)], idxbuf.at[0],
                                  sem_s0).wait()

        @pl.when(jnp.logical_and(k >= 2, kb == 1))
        def _():
            pltpu.make_async_copy(frag.at[pl.ds(0, C)], idxbuf.at[0],
                                  sem_s1).wait()

        def _row(j, dv):
            for jj in range(8):
                v = idxbuf[kb, pl.ds(j * 128 + jj * 16, 16)]
                loc = v - base
                m = jnp.logical_and(loc >= 0, loc < H)
                dv = (dv + 61) & (DUMMY - 1)
                word = jnp.where(m, loc >> 1, HW + dv)
                val = jnp.where((v & 1) == 1, 1 << 16, 1)
                cbuf[kb, j, pl.ds(jj * 16, 16)] = word
                vbuf[kb, j, pl.ds(jj * 16, 16)] = val
            return dv
        dvec = lax.fori_loop(0, ROWS, _row, dvec)

        def _fire0(j, _):
            pltpu.async_copy(vbuf.at[kb, j], hist.at[cbuf.at[kb, j]],
                             sem_s0, add=True)
            return 0

        def _fire1(j, _):
            pltpu.async_copy(vbuf.at[kb, j], hist.at[cbuf.at[kb, j]],
                             sem_s1, add=True)
            return 0

        @pl.when(kb == 0)
        def _():
            lax.fori_loop(0, ROWS, _fire0, 0)

        @pl.when(kb == 1)
        def _():
            lax.fori_loop(0, ROWS, _fire1, 0)
        return dvec

    lax.fori_loop(0, K, _chunk, dvec0)

    pltpu.make_async_copy(frag.at[pl.ds(0, C)], idxbuf.at[0], sem_s0).wait()
    pltpu.make_async_copy(frag.at[pl.ds(0, C)], idxbuf.at[0], sem_s1).wait()
    plsc.subcore_barrier()

    obase = 2 * (c * HW + hoff)
    pltpu.async_copy(hist.at[pl.ds(hoff, C)], idxbuf.at[0], sem_in)

    def _unpack(src_plane, dst):
        def _u(t, _):
            w = idxbuf[src_plane, pl.ds(t * 16, 16)]
            dst[pl.ds(t * 32, 32)] = plsc.bitcast(w, jnp.int16)
            return 0
        lax.fori_loop(0, C // 16, _u, 0)

    def _dump2(i2, _):
        i0 = 2 * i2
        pltpu.make_async_copy(hist.at[pl.ds(hoff, C)], idxbuf.at[0],
                              sem_in).wait()
        pltpu.async_copy(hist.at[pl.ds(hoff + (i0 + 1) * C, C)],
                         idxbuf.at[1], sem_in)

        @pl.when(i2 >= 1)
        def _():
            pltpu.make_async_copy(pbufa, count_out.at[pl.ds(obase, 2 * C)],
                                  sem_g).wait()
        _unpack(0, pbufa)
        pltpu.async_copy(pbufa, count_out.at[pl.ds(obase + 2 * i0 * C, 2 * C)],
                         sem_g)

        pltpu.make_async_copy(hist.at[pl.ds(hoff, C)], idxbuf.at[0],
                              sem_in).wait()

        @pl.when(i0 + 2 < HPW // C)
        def _():
            pltpu.async_copy(hist.at[pl.ds(hoff + (i0 + 2) * C, C)],
                             idxbuf.at[0], sem_in)

        @pl.when(i2 >= 1)
        def _():
            pltpu.make_async_copy(pbufb, count_out.at[pl.ds(obase, 2 * C)],
                                  sem_g).wait()
        _unpack(1, pbufb)
        pltpu.async_copy(pbufb,
                         count_out.at[pl.ds(obase + 2 * (i0 + 1) * C, 2 * C)],
                         sem_g)
        return 0
    lax.fori_loop(0, HPW // C // 2, _dump2, 0)
    pltpu.make_async_copy(pbufa, count_out.at[pl.ds(obase, 2 * C)],
                          sem_g).wait()
    pltpu.make_async_copy(pbufb, count_out.at[pl.ds(obase, 2 * C)],
                          sem_g).wait()

    @pl.when(jnp.logical_and(c == 0, s == 1))
    def _():
        pltpu.sync_copy(regions2, cbuf.at[0, pl.ds(0, 8)])
        for j in range(8):
            pltpu.async_copy(baseline.at[cbuf.at[0, j]],
                             zbuf.at[pl.ds(j * 128, 128)], sem_s0)
        pltpu.make_async_copy(baseline.at[pl.ds(0, 1024)],
                              zbuf.at[pl.ds(0, 1024)], sem_s0).wait()
        pltpu.sync_copy(zbuf.at[pl.ds(0, 1024)], rb_out)

    @pl.when(jnp.logical_and(c == 1, s == 1))
    def _():
        pltpu.sync_copy(cells2, cbuf.at[0, pl.ds(0, 32)])
        for j in range(32):
            pltpu.async_copy(lib.at[cbuf.at[0, j]],
                             zbuf.at[pl.ds(j * 128, 128)], sem_s0)
        pltpu.make_async_copy(lib.at[pl.ds(0, C)], zbuf, sem_s0).wait()
        pltpu.sync_copy(zbuf, cl_out)


@functools.cache
def _sc_histogram():
    return functools.partial(
        pl.kernel,
        out_type=(
            jax.ShapeDtypeStruct((NB,), jnp.int16),
            jax.ShapeDtypeStruct((N_REGIONS,), jnp.float32),
            jax.ShapeDtypeStruct((N_CELLS,), jnp.float32),
        ),
        mesh=plsc.VectorSubcoreMesh(core_axis_name="c", subcore_axis_name="s",
                                    num_cores=NC, num_subcores=NS),
        compiler_params=pltpu.CompilerParams(needs_layout_passes=False),
        scratch_types=[
            pltpu.VMEM_SHARED((HW + DUMMY,), jnp.int32),
            pltpu.VMEM((2, C), jnp.int32),
            pltpu.VMEM((2, ROWS, 128), jnp.int32),
            pltpu.VMEM((2, ROWS, 128), jnp.int32),
            pltpu.VMEM((C,), jnp.float32),
            pltpu.VMEM((2 * C,), jnp.int16),
            pltpu.VMEM((2 * C,), jnp.int16),
            pltpu.SemaphoreType.DMA,
            pltpu.SemaphoreType.DMA,
            pltpu.SemaphoreType.DMA,
            pltpu.SemaphoreType.DMA,
        ],
    )(_sc_body)


BLK = 1024


def _gammaln1p(c):
    x = c + 1.0
    p = (x * (x + 1.0) * (x + 2.0) * (x + 3.0)
         * (x + 4.0) * (x + 5.0) * (x + 6.0) * (x + 7.0))
    z = x + 8.0
    zi = 1.0 / z
    zi2 = zi * zi
    series = zi * (1.0 / 12.0 - zi2 * (1.0 / 360.0 - zi2 * (1.0 / 1260.0)))
    lg = (z - 0.5) * jnp.log(z) - z + 0.9189385332046727 + series
    return lg - jnp.log(p)


def _tc_body(count_ref, cl_ref, rb_ref, out_ref):
    cnt = count_ref[...].astype(jnp.float32)
    logits = cl_ref[...] + rb_ref[...]
    rate = jnp.exp(logits)
    out_ref[...] = cnt * logits - rate - _gammaln1p(cnt)


_tc_likelihood = pl.pallas_call(
    _tc_body,
    grid=(N_CELLS // BLK,),
    in_specs=[
        pl.BlockSpec((BLK, N_REGIONS), lambda i: (i, 0)),
        pl.BlockSpec((BLK, 1), lambda i: (i, 0)),
        pl.BlockSpec((1, N_REGIONS), lambda i: (0, 0)),
    ],
    out_specs=pl.BlockSpec((BLK, N_REGIONS), lambda i: (i, 0)),
    out_shape=jax.ShapeDtypeStruct((N_CELLS, N_REGIONS), jnp.float32),
)


def kernel(local_cellxregion_ix, regions_oi, cells_oi, baseline_weight, lib):
    regions2 = regions_oi.reshape(8, 128)
    cells2 = cells_oi.reshape(32, 128)
    baseline_flat = baseline_weight.reshape(-1)
    count, rb, cl = _sc_histogram()(local_cellxregion_ix, regions2, cells2,
                                    baseline_flat, lib)
    count2 = count.reshape(N_CELLS, N_REGIONS)
    cl2 = cl.reshape(N_CELLS, 1)
    rb2 = rb.reshape(1, N_REGIONS)
    return _tc_likelihood(count2, cl2, rb2)


# speedup vs baseline: 2.6450x; 1.0001x over previous
import functools

import jax
import jax.numpy as jnp
from jax import lax
from jax.experimental import pallas as pl
from jax.experimental.pallas import tpu as pltpu
from jax.experimental.pallas import tpu_sc as plsc

NF = 8388608
N_CELLS = 4096
N_REGIONS = 1024
NB = N_CELLS * N_REGIONS

NC = 2
NS = 16

H = NB // NC
HW = H // 2
DUMMY = 65536
FPT = NF // NS
C = 4096
K = FPT // C
ROWS = C // 128
HPW = HW // NS


def _sc_body(frag, regions2, cells2, baseline, lib,
             count_out, rb_out, cl_out,
             hist, idxbuf, cbuf, vbuf, zbuf, pbufa, pbufb,
             sem_in, sem_s0, sem_s1, sem_g):
    c = lax.axis_index("c")
    s = lax.axis_index("s")
    base = c * H

    zvi = jnp.zeros((16,), jnp.int32)
    iv = lax.iota(jnp.int32, 16)

    def _z16(i, _):
        idxbuf[0, pl.ds(i * 16, 16)] = zvi
        return 0
    lax.fori_loop(0, C // 16, _z16, 0)

    hoff = s * HPW

    def _zh(i, _):
        pltpu.sync_copy(idxbuf.at[0], hist.at[pl.ds(hoff + i * C, C)])
        return 0
    lax.fori_loop(0, HPW // C, _zh, 0)
    plsc.subcore_barrier()

    dvec0 = (iv * 4096 + s * 256) & (DUMMY - 1)
    pltpu.async_copy(frag.at[pl.ds(s * FPT, C)], idxbuf.at[0], sem_in)

    def _chunk(k, dvec):
        kb = lax.rem(k, 2)
        pltpu.make_async_copy(frag.at[pl.ds(0, C)], idxbuf.at[0],
                              sem_in).wait()

        @pl.when(k + 1 < K)
        def _():
            pltpu.async_copy(frag.at[pl.ds(s * FPT + (k + 1) * C, C)],
                             idxbuf.at[1 - kb], sem_in)

        @pl.when(jnp.logical_and(k >= 2, kb == 0))
        def _():
            pltpu.make_async_copy(frag.at[pl.ds(0, C)], idxbuf.at[0],
                                  sem_s0).wait()

        @pl.when(jnp.logical_and(k >= 2, kb == 1))
        def _():
            pltpu.make_async_copy(frag.at[pl.ds(0, C)], idxbuf.at[0],
                                  sem_s1).wait()

        def _row(j, dv):
            for jj in range(8):
                v = idxbuf[kb, pl.ds(j * 128 + jj * 16, 16)]
                loc = v - base
                m = jnp.logical_and(loc >= 0, loc < H)
                dv = (dv + 61) & (DUMMY - 1)
                word = jnp.where(m, loc >> 1, HW + dv)
                val = jnp.where((v & 1) == 1, 1 << 16, 1)
                cbuf[kb, j, pl.ds(jj * 16, 16)] = word
                vbuf[kb, j, pl.ds(jj * 16, 16)] = val
            return dv
        dvec = lax.fori_loop(0, ROWS, _row, dvec)

        def _fire0(j, _):
            pltpu.async_copy(vbuf.at[kb, j], hist.at[cbuf.at[kb, j]],
                             sem_s0, add=True)
            return 0

        def _fire1(j, _):
            pltpu.async_copy(vbuf.at[kb, j], hist.at[cbuf.at[kb, j]],
                             sem_s1, add=True)
            return 0

        @pl.when(kb == 0)
        def _():
            lax.fori_loop(0, ROWS, _fire0, 0)

        @pl.when(kb == 1)
        def _():
            lax.fori_loop(0, ROWS, _fire1, 0)
        return dvec

    lax.fori_loop(0, K, _chunk, dvec0)

    pltpu.make_async_copy(frag.at[pl.ds(0, C)], idxbuf.at[0], sem_s0).wait()
    pltpu.make_async_copy(frag.at[pl.ds(0, C)], idxbuf.at[0], sem_s1).wait()
    plsc.subcore_barrier()

    obase = 2 * (c * HW + hoff)
    pltpu.async_copy(hist.at[pl.ds(hoff, C)], idxbuf.at[0], sem_in)

    def _unpack(src_plane, dst):
        def _u(t, _):
            w = idxbuf[src_plane, pl.ds(t * 16, 16)]
            dst[pl.ds(t * 32, 32)] = plsc.bitcast(w, jnp.int16)
            return 0
        lax.fori_loop(0, C // 16, _u, 0)

    def _dump2(i2, _):
        i0 = 2 * i2
        pltpu.make_async_copy(hist.at[pl.ds(hoff, C)], idxbuf.at[0],
                              sem_in).wait()
        pltpu.async_copy(hist.at[pl.ds(hoff + (i0 + 1) * C, C)],
                         idxbuf.at[1], sem_in)

        @pl.when(i2 >= 1)
        def _():
            pltpu.make_async_copy(pbufa, count_out.at[pl.ds(obase, 2 * C)],
                                  sem_g).wait()
        _unpack(0, pbufa)
        pltpu.async_copy(pbufa, count_out.at[pl.ds(obase + 2 * i0 * C, 2 * C)],
                         sem_g)

        pltpu.make_async_copy(hist.at[pl.ds(hoff, C)], idxbuf.at[0],
                              sem_in).wait()

        @pl.when(i0 + 2 < HPW // C)
        def _():
            pltpu.async_copy(hist.at[pl.ds(hoff + (i0 + 2) * C, C)],
                             idxbuf.at[0], sem_in)

        @pl.when(i2 >= 1)
        def _():
            pltpu.make_async_copy(pbufb, count_out.at[pl.ds(obase, 2 * C)],
                                  sem_g).wait()
        _unpack(1, pbufb)
        pltpu.async_copy(pbufb,
                         count_out.at[pl.ds(obase + 2 * (i0 + 1) * C, 2 * C)],
                         sem_g)
        return 0
    lax.fori_loop(0, HPW // C // 2, _dump2, 0)
    pltpu.make_async_copy(pbufa, count_out.at[pl.ds(obase, 2 * C)],
                          sem_g).wait()
    pltpu.make_async_copy(pbufb, count_out.at[pl.ds(obase, 2 * C)],
                          sem_g).wait()

    @pl.when(jnp.logical_and(c == 0, s == 1))
    def _():
        pltpu.sync_copy(regions2, cbuf.at[0, pl.ds(0, 8)])
        for j in range(8):
            pltpu.async_copy(baseline.at[cbuf.at[0, j]],
                             zbuf.at[pl.ds(j * 128, 128)], sem_s0)
        pltpu.make_async_copy(baseline.at[pl.ds(0, 1024)],
                              zbuf.at[pl.ds(0, 1024)], sem_s0).wait()
        pltpu.sync_copy(zbuf.at[pl.ds(0, 1024)], rb_out)

    @pl.when(jnp.logical_and(c == 1, s == 1))
    def _():
        pltpu.sync_copy(cells2, cbuf.at[0, pl.ds(0, 32)])
        for j in range(32):
            pltpu.async_copy(lib.at[cbuf.at[0, j]],
                             zbuf.at[pl.ds(j * 128, 128)], sem_s0)
        pltpu.make_async_copy(lib.at[pl.ds(0, C)], zbuf, sem_s0).wait()
        pltpu.sync_copy(zbuf, cl_out)


@functools.cache
def _sc_histogram():
    return functools.partial(
        pl.kernel,
        out_type=(
            jax.ShapeDtypeStruct((NB,), jnp.int16),
            jax.ShapeDtypeStruct((N_REGIONS,), jnp.float32),
            jax.ShapeDtypeStruct((N_CELLS,), jnp.float32),
        ),
        mesh=plsc.VectorSubcoreMesh(core_axis_name="c", subcore_axis_name="s",
                                    num_cores=NC, num_subcores=NS),
        compiler_params=pltpu.CompilerParams(needs_layout_passes=False),
        scratch_types=[
            pltpu.VMEM_SHARED((HW + DUMMY,), jnp.int32),
            pltpu.VMEM((2, C), jnp.int32),
            pltpu.VMEM((2, ROWS, 128), jnp.int32),
            pltpu.VMEM((2, ROWS, 128), jnp.int32),
            pltpu.VMEM((C,), jnp.float32),
            pltpu.VMEM((2 * C,), jnp.int16),
            pltpu.VMEM((2 * C,), jnp.int16),
            pltpu.SemaphoreType.DMA,
            pltpu.SemaphoreType.DMA,
            pltpu.SemaphoreType.DMA,
            pltpu.SemaphoreType.DMA,
        ],
    )(_sc_body)


BLK = 1024


def _gammaln1p(c):
    x = c + 1.0
    p = (x * (x + 1.0) * (x + 2.0) * (x + 3.0)
         * (x + 4.0) * (x + 5.0) * (x + 6.0) * (x + 7.0))
    z = x + 8.0
    zi = 1.0 / z
    zi2 = zi * zi
    series = zi * (1.0 / 12.0 - zi2 * (1.0 / 360.0 - zi2 * (1.0 / 1260.0)))
    lg = (z - 0.5) * jnp.log(z) - z + 0.9189385332046727 + series
    return lg - jnp.log(p)


def _tc_body(count_ref, cl_ref, rb_ref, out_ref):
    cnt = count_ref[...].astype(jnp.float32)
    cl = cl_ref[...]
    rb = rb_ref[...]
    logits = cl + rb
    rate = jnp.exp(cl) * jnp.exp(rb)
    out_ref[...] = cnt * logits - rate - _gammaln1p(cnt)


_tc_likelihood = pl.pallas_call(
    _tc_body,
    grid=(N_CELLS // BLK,),
    in_specs=[
        pl.BlockSpec((BLK, N_REGIONS), lambda i: (i, 0)),
        pl.BlockSpec((BLK, 1), lambda i: (i, 0)),
        pl.BlockSpec((1, N_REGIONS), lambda i: (0, 0)),
    ],
    out_specs=pl.BlockSpec((BLK, N_REGIONS), lambda i: (i, 0)),
    out_shape=jax.ShapeDtypeStruct((N_CELLS, N_REGIONS), jnp.float32),
)


def kernel(local_cellxregion_ix, regions_oi, cells_oi, baseline_weight, lib):
    regions2 = regions_oi.reshape(8, 128)
    cells2 = cells_oi.reshape(32, 128)
    baseline_flat = baseline_weight.reshape(-1)
    count, rb, cl = _sc_histogram()(local_cellxregion_ix, regions2, cells2,
                                    baseline_flat, lib)
    count2 = count.reshape(N_CELLS, N_REGIONS)
    cl2 = cl.reshape(N_CELLS, 1)
    rb2 = rb.reshape(1, N_REGIONS)
    return _tc_likelihood(count2, cl2, rb2)


# speedup vs baseline: 2.7736x; 1.0486x over previous
import functools

import jax
import jax.numpy as jnp
from jax import lax
from jax.experimental import pallas as pl
from jax.experimental.pallas import tpu as pltpu
from jax.experimental.pallas import tpu_sc as plsc

NF = 8388608
N_CELLS = 4096
N_REGIONS = 1024
NB = N_CELLS * N_REGIONS

NC = 2
NS = 16

H = NB // NC
HW = H // 2
DUMMY = 65536
FPT = NF // NS
C = 4096
K = FPT // C
ROWS = C // 128
HPW = HW // NS
TBL = 128


def _sc_body(frag, regions2, cells2, baseline, lib, table,
             out_hbm,
             hist, idxbuf, cbuf, vbuf, tbl, clbuf, eclbuf, rbbuf,
             erbe, erbo, osta, ostb,
             sem_in, sem_s0, sem_s1, sem_g):
    c = lax.axis_index("c")
    s = lax.axis_index("s")
    base = c * H

    zvi = jnp.zeros((16,), jnp.int32)
    iv = lax.iota(jnp.int32, 16)

    def _z16(i, _):
        idxbuf[0, pl.ds(i * 16, 16)] = zvi
        return 0
    lax.fori_loop(0, C // 16, _z16, 0)

    hoff = s * HPW

    def _zh(i, _):
        pltpu.sync_copy(idxbuf.at[0], hist.at[pl.ds(hoff + i * C, C)])
        return 0
    lax.fori_loop(0, HPW // C, _zh, 0)
    plsc.subcore_barrier()

    dvec0 = (iv * 4096 + s * 256) & (DUMMY - 1)
    pltpu.async_copy(frag.at[pl.ds(s * FPT, C)], idxbuf.at[0], sem_in)

    def _chunk(k, dvec):
        kb = lax.rem(k, 2)
        pltpu.make_async_copy(frag.at[pl.ds(0, C)], idxbuf.at[0],
                              sem_in).wait()

        @pl.when(k + 1 < K)
        def _():
            pltpu.async_copy(frag.at[pl.ds(s * FPT + (k + 1) * C, C)],
                             idxbuf.at[1 - kb], sem_in)

        @pl.when(jnp.logical_and(k >= 2, kb == 0))
        def _():
            pltpu.make_async_copy(frag.at[pl.ds(0, C)], idxbuf.at[0],
                                  sem_s0).wait()

        @pl.when(jnp.logical_and(k >= 2, kb == 1))
        def _():
            pltpu.make_async_copy(frag.at[pl.ds(0, C)], idxbuf.at[0],
                                  sem_s1).wait()

        def _row(j, dv):
            for jj in range(8):
                v = idxbuf[kb, pl.ds(j * 128 + jj * 16, 16)]
                loc = v - base
                m = jnp.logical_and(loc >= 0, loc < H)
                dv = (dv + 61) & (DUMMY - 1)
                word = jnp.where(m, loc >> 1, HW + dv)
                val = jnp.where((v & 1) == 1, 1 << 16, 1)
                cbuf[kb, j, pl.ds(jj * 16, 16)] = word
                vbuf[kb, j, pl.ds(jj * 16, 16)] = val
            return dv
        dvec = lax.fori_loop(0, ROWS, _row, dvec)

        def _fire0(j, _):
            pltpu.async_copy(vbuf.at[kb, j], hist.at[cbuf.at[kb, j]],
                             sem_s0, add=True)
            return 0

        def _fire1(j, _):
            pltpu.async_copy(vbuf.at[kb, j], hist.at[cbuf.at[kb, j]],
                             sem_s1, add=True)
            return 0

        @pl.when(kb == 0)
        def _():
            lax.fori_loop(0, ROWS, _fire0, 0)

        @pl.when(kb == 1)
        def _():
            lax.fori_loop(0, ROWS, _fire1, 0)
        return dvec

    lax.fori_loop(0, K, _chunk, dvec0)

    pltpu.make_async_copy(frag.at[pl.ds(0, C)], idxbuf.at[0], sem_s0).wait()
    pltpu.make_async_copy(frag.at[pl.ds(0, C)], idxbuf.at[0], sem_s1).wait()

    pltpu.sync_copy(table, tbl)
    pltpu.sync_copy(cells2.at[16 * c + s], cbuf.at[0, 0])
    pltpu.async_copy(lib.at[cbuf.at[0, 0]], clbuf, sem_g)
    pltpu.sync_copy(regions2, cbuf.at[1, pl.ds(0, 8)])
    for j in range(8):
        pltpu.async_copy(baseline.at[cbuf.at[1, j]],
                         rbbuf.at[pl.ds(j * 128, 128)], sem_g)
    pltpu.make_async_copy(baseline.at[pl.ds(0, 1024)], rbbuf, sem_g).wait()
    pltpu.make_async_copy(lib.at[pl.ds(0, 128)], clbuf, sem_g).wait()

    def _ecl(t, _):
        eclbuf[pl.ds(t * 16, 16)] = jnp.exp(clbuf[pl.ds(t * 16, 16)])
        return 0
    lax.fori_loop(0, 8, _ecl, 0)

    def _deint(t, _):
        ev = plsc.load_gather(rbbuf, [32 * t + 2 * iv])
        ov = plsc.load_gather(rbbuf, [32 * t + 2 * iv + 1])
        erbe[pl.ds(t * 16, 16)] = ev
        erbo[pl.ds(t * 16, 16)] = ov
        return 0
    lax.fori_loop(0, 32, _deint, 0)

    def _eexp(t, _):
        erbe[pl.ds(t * 16, 16)] = jnp.exp(erbe[pl.ds(t * 16, 16)])
        erbo[pl.ds(t * 16, 16)] = jnp.exp(erbo[pl.ds(t * 16, 16)])
        return 0
    lax.fori_loop(0, 32, _eexp, 0)

    plsc.subcore_barrier()

    dv2 = 2 * iv
    obase = 2 * (c * HW + hoff)

    def _like(src_plane, ost):
        def _rr(rr, _):
            cell = jnp.full((16,), 0, jnp.int32) + rr
            clv = plsc.load_gather(clbuf, [cell])
            eclv = plsc.load_gather(eclbuf, [cell])

            def _t(t, _):
                w = idxbuf[src_plane, pl.ds((rr & 7) * 512 + t * 16, 16)]
                lo = w & 0xFFFF
                hi = (w >> 16) & 0xFFFF
                rbev = plsc.load_gather(rbbuf, [dv2 + 32 * t])
                rbov = plsc.load_gather(rbbuf, [dv2 + 32 * t + 1])
                le = clv + rbev
                lod = clv + rbov
                re = eclv * erbe[pl.ds(t * 16, 16)]
                ro = eclv * erbo[pl.ds(t * 16, 16)]
                ge = plsc.load_gather(tbl, [jnp.minimum(lo, TBL - 1)])
                go = plsc.load_gather(tbl, [jnp.minimum(hi, TBL - 1)])
                fe = lo.astype(jnp.float32)
                fo = hi.astype(jnp.float32)
                lle = fe * le - re - ge
                llo = fo * lod - ro - go
                pbase = (rr & 7) * 1024 + 32 * t
                plsc.store_scatter(ost, [pbase + dv2], lle)
                plsc.store_scatter(ost, [pbase + dv2 + 1], llo)
                return 0
            lax.fori_loop(0, 32, _t, 0)
            return 0
        return _rr

    pltpu.async_copy(hist.at[pl.ds(hoff, C)], idxbuf.at[0], sem_in)

    def _dump2(i2, _):
        i0 = 2 * i2
        pltpu.make_async_copy(hist.at[pl.ds(hoff, C)], idxbuf.at[0],
                              sem_in).wait()
        pltpu.async_copy(hist.at[pl.ds(hoff + (i0 + 1) * C, C)],
                         idxbuf.at[1], sem_in)

        @pl.when(i2 >= 1)
        def _():
            pltpu.make_async_copy(osta, out_hbm.at[pl.ds(obase, 2 * C)],
                                  sem_g).wait()
        lax.fori_loop(8 * i0, 8 * i0 + 8, _like(0, osta), 0)
        pltpu.async_copy(osta, out_hbm.at[pl.ds(obase + 2 * i0 * C, 2 * C)],
                         sem_g)

        pltpu.make_async_copy(hist.at[pl.ds(hoff, C)], idxbuf.at[0],
                              sem_in).wait()

        @pl.when(i0 + 2 < HPW // C)
        def _():
            pltpu.async_copy(hist.at[pl.ds(hoff + (i0 + 2) * C, C)],
                             idxbuf.at[0], sem_in)

        @pl.when(i2 >= 1)
        def _():
            pltpu.make_async_copy(ostb, out_hbm.at[pl.ds(obase, 2 * C)],
                                  sem_g).wait()
        lax.fori_loop(8 * (i0 + 1), 8 * (i0 + 1) + 8, _like(1, ostb), 0)
        pltpu.async_copy(ostb,
                         out_hbm.at[pl.ds(obase + 2 * (i0 + 1) * C, 2 * C)],
                         sem_g)
        return 0
    lax.fori_loop(0, HPW // C // 2, _dump2, 0)
    pltpu.make_async_copy(osta, out_hbm.at[pl.ds(obase, 2 * C)], sem_g).wait()
    pltpu.make_async_copy(ostb, out_hbm.at[pl.ds(obase, 2 * C)], sem_g).wait()


@functools.cache
def _sc_kernel():
    return functools.partial(
        pl.kernel,
        out_type=jax.ShapeDtypeStruct((NB,), jnp.float32),
        mesh=plsc.VectorSubcoreMesh(core_axis_name="c", subcore_axis_name="s",
                                    num_cores=NC, num_subcores=NS),
        compiler_params=pltpu.CompilerParams(needs_layout_passes=False),
        scratch_types=[
            pltpu.VMEM_SHARED((HW + DUMMY,), jnp.int32),
            pltpu.VMEM((2, C), jnp.int32),
            pltpu.VMEM((2, ROWS, 128), jnp.int32),
            pltpu.VMEM((2, ROWS, 128), jnp.int32),
            pltpu.VMEM((TBL,), jnp.float32),
            pltpu.VMEM((128,), jnp.float32),
            pltpu.VMEM((128,), jnp.float32),
            pltpu.VMEM((1024,), jnp.float32),
            pltpu.VMEM((512,), jnp.float32),
            pltpu.VMEM((512,), jnp.float32),
            pltpu.VMEM((2 * C,), jnp.float32),
            pltpu.VMEM((2 * C,), jnp.float32),
            pltpu.SemaphoreType.DMA,
            pltpu.SemaphoreType.DMA,
            pltpu.SemaphoreType.DMA,
            pltpu.SemaphoreType.DMA,
        ],
    )(_sc_body)


def kernel(local_cellxregion_ix, regions_oi, cells_oi, baseline_weight, lib):
    regions2 = regions_oi.reshape(8, 128)
    cells2 = cells_oi.reshape(32, 128)
    baseline_flat = baseline_weight.reshape(-1)
    table = jax.scipy.special.gammaln(jnp.arange(TBL, dtype=jnp.float32)
                                      + 1.0)
    ll = _sc_kernel()(local_cellxregion_ix, regions2, cells2,
                      baseline_flat, lib, table)
    return ll.reshape(N_CELLS, N_REGIONS)
